# baseline (speedup 1.0000x reference)
import jax
import jax.numpy as jnp
from jax import lax
from jax.experimental import pallas as pl
from jax.experimental.pallas import tpu as pltpu

_NCLS = 8
_PACK = 4
_BLK = 2048


def _body(lab_ref, stab_ref, ttab_ref, x_ref, o_ref):
    lab = lab_ref[...]
    code = (lab[:, 0:1] + (lab[:, 1:2] << 3) + (lab[:, 2:3] << 6)
            + (lab[:, 3:4] << 9))
    k = lax.broadcasted_iota(jnp.int32, (1, _PACK * _NCLS), 1)
    oh = ((code >> (3 * (k >> 3))) & 7) == (k & 7)
    ohf = oh.astype(jnp.float32)
    rs = jnp.dot(ohf, stab_ref[...], preferred_element_type=jnp.float32,
                 precision=lax.Precision.HIGHEST)
    rb = jnp.dot(ohf, ttab_ref[...], preferred_element_type=jnp.float32,
                 precision=lax.Precision.HIGHEST)
    o_ref[...] = x_ref[...] * rs + rb


def kernel(x, labels, scale, shift):
    n, d = x.shape
    nr = n // _PACK
    xv = x.reshape(nr, _PACK * d)
    lab4 = labels.reshape(nr, _PACK)
    eye = jnp.eye(_PACK, dtype=scale.dtype)
    stab = (eye[:, None, :, None] * scale[None, :, None, :]).reshape(
        _PACK * _NCLS, _PACK * d)
    ttab = (eye[:, None, :, None] * shift[None, :, None, :]).reshape(
        _PACK * _NCLS, _PACK * d)
    grid = (nr // _BLK,)
    out = pl.pallas_call(
        _body,
        grid=grid,
        in_specs=[
            pl.BlockSpec((_BLK, _PACK), lambda i: (i, 0)),
            pl.BlockSpec((_PACK * _NCLS, _PACK * d), lambda i: (0, 0)),
            pl.BlockSpec((_PACK * _NCLS, _PACK * d), lambda i: (0, 0)),
            pl.BlockSpec((_BLK, _PACK * d), lambda i: (i, 0)),
        ],
        out_specs=pl.BlockSpec((_BLK, _PACK * d), lambda i: (i, 0)),
        out_shape=jax.ShapeDtypeStruct((nr, _PACK * d), x.dtype),
        compiler_params=pltpu.CompilerParams(
            dimension_semantics=("arbitrary",),
        ),
    )(lab4, stab, ttab, xv)
    return out.reshape(n, d)
